# baseline (speedup 1.0000x reference)
import functools

import jax
import jax.numpy as jnp
from jax import lax
from jax.experimental import pallas as pl
from jax.experimental.pallas import tpu as pltpu
from jax.experimental.pallas import tpu_sc as plsc

N_SPARSE = 26
N_DENSE = 13
VOCAB = 100000
EMB = 16
LAYER_NUM = 3
DIM = N_SPARSE * EMB + N_DENSE
NF = N_SPARSE * EMB

NW = 32
FPC = 13


@functools.cache
def _make_gather_t(b_total: int, nfields: int):
    assert b_total % NW == 0
    bs = b_total // NW
    rows = nfields * EMB

    mesh = plsc.VectorSubcoreMesh(core_axis_name="c", subcore_axis_name="s")

    @functools.partial(
        pl.kernel,
        mesh=mesh,
        out_type=jax.ShapeDtypeStruct((rows, b_total), jnp.float32),
        scratch_types=[
            pltpu.VMEM((nfields, bs), jnp.int32),
            pltpu.VMEM((rows, bs), jnp.float32),
            pltpu.SemaphoreType.DMA,
        ],
        compiler_params=pltpu.CompilerParams(use_tc_tiling_on_sc=False),
    )
    def gather(table_hbm, idx_hbm, out_hbm, idx_v, g_v, sem):
        wid = lax.axis_index("s") * 2 + lax.axis_index("c")
        b0 = wid * bs
        pltpu.sync_copy(idx_hbm.at[:, pl.ds(b0, bs)], idx_v)

        def body(f, _):
            for e in range(EMB):
                pltpu.async_copy(
                    table_hbm.at[f * EMB + e].at[idx_v.at[f]],
                    g_v.at[f * EMB + e], sem)
            @pl.when(f > 0)
            def _():
                pltpu.make_async_copy(
                    table_hbm.at[pl.ds(0, EMB), pl.ds(0, bs)],
                    g_v.at[pl.ds(0, EMB)], sem).wait()
            return 0

        lax.fori_loop(0, nfields, body, 0)
        pltpu.make_async_copy(
            table_hbm.at[pl.ds(0, EMB), pl.ds(0, bs)],
            g_v.at[pl.ds(0, EMB)], sem).wait()
        pltpu.sync_copy(g_v, out_hbm.at[:, pl.ds(b0, bs)])

    return gather


def _dense_t_body(embt_ref, dent_ref, cw_ref, cb_ref, w1t_ref,
                  b1_ref, w2t_ref, b2_ref, w3t_ref, b3_ref, wft_ref, bf_ref,
                  out_ref):
    f32 = jnp.float32
    b16 = jnp.bfloat16
    xt = jnp.concatenate([embt_ref[...], dent_ref[...]],
                         axis=0)
    xt16 = xt.astype(b16)
    cw = cw_ref[...]
    xl = xt
    for i in range(LAYER_NUM):
        s = jnp.dot(cw[i:i + 1, :], xl.astype(b16),
                    preferred_element_type=f32)
        xl = (xt * s + cb_ref[i][:, None]) + xl
    ht16 = xt16
    for wt_ref, b_ref in ((w1t_ref, b1_ref), (w2t_ref, b2_ref),
                          (w3t_ref, b3_ref)):
        h = jnp.dot(wt_ref[...], ht16, preferred_element_type=f32)
        ht16 = jnp.maximum(h + b_ref[...], 0.0).astype(b16)
    cat = jnp.concatenate([xl.astype(b16), ht16], axis=0)
    t = jnp.dot(wft_ref[...], cat, preferred_element_type=f32) + bf_ref[...]
    out_ref[...] = jax.nn.sigmoid(t)


def _dense_t_forward(embt, dent, cw2, cb2, w1, b1, w2, b2, w3, b3,
                     wf, bf, blk: int):
    b16 = jnp.bfloat16
    b_total = embt.shape[1]
    h0, h1, h2 = w1.shape[1], w2.shape[1], w3.shape[1]
    full = lambda *shape: pl.BlockSpec(shape, lambda i: (0,) * len(shape))
    return pl.pallas_call(
        _dense_t_body,
        grid=(b_total // blk,),
        in_specs=[
            pl.BlockSpec((NF, blk), lambda i: (0, i)),
            pl.BlockSpec((N_DENSE, blk), lambda i: (0, i)),
            full(LAYER_NUM, DIM),
            full(LAYER_NUM, DIM),
            full(h0, DIM),
            full(h0, 1),
            full(h1, h0),
            full(h1, 1),
            full(h2, h1),
            full(h2, 1),
            full(1, DIM + h2),
            full(1, 1),
        ],
        out_specs=pl.BlockSpec((1, blk), lambda i: (0, i)),
        out_shape=jax.ShapeDtypeStruct((1, b_total), jnp.float32),
        compiler_params=pltpu.CompilerParams(
            dimension_semantics=("arbitrary",)),
    )(embt, dent, cw2.astype(b16), cb2,
      w1.astype(b16).T, b1.reshape(h0, 1),
      w2.astype(b16).T, b2.reshape(h1, 1),
      w3.astype(b16).T, b3.reshape(h2, 1),
      wf.astype(b16).T, bf.reshape(1, 1))


def kernel(inputs, E, cw, cb, W1, b1, W2, b2, W3, b3, Wf, bf):
    b_total = inputs.shape[0]
    int_ = jnp.swapaxes(inputs, 0, 1)
    dent = int_[N_SPARSE:]
    idxt = int_[:N_SPARSE].astype(jnp.int32)
    table = jnp.swapaxes(E, 1, 2).reshape(NF, VOCAB)
    hb = b_total // 2
    gather = _make_gather_t(hb, N_SPARSE)
    outs = []
    for c in range(2):
        embt_c = gather(table, idxt[:, c * hb:(c + 1) * hb])
        outs.append(_dense_t_forward(
            embt_c, dent[:, c * hb:(c + 1) * hb], cw[..., 0], cb[..., 0],
            W1, b1, W2, b2, W3, b3, Wf, bf, blk=2048))
    return jnp.concatenate(outs, axis=1).reshape(b_total, 1)


# speedup vs baseline: 1.0301x; 1.0301x over previous
import functools

import jax
import jax.numpy as jnp
from jax import lax
from jax.experimental import pallas as pl
from jax.experimental.pallas import tpu as pltpu
from jax.experimental.pallas import tpu_sc as plsc

N_SPARSE = 26
N_DENSE = 13
VOCAB = 100000
EMB = 16
LAYER_NUM = 3
DIM = N_SPARSE * EMB + N_DENSE
NF = N_SPARSE * EMB

NW = 32
FPC = 13


@functools.cache
def _make_gather_t(b_total: int):
    assert b_total % NW == 0
    bs = b_total // NW
    rows = FPC * EMB

    mesh = plsc.VectorSubcoreMesh(core_axis_name="c", subcore_axis_name="s")

    @functools.partial(
        pl.kernel,
        mesh=mesh,
        out_type=jax.ShapeDtypeStruct((NF, b_total), jnp.float32),
        scratch_types=[
            pltpu.VMEM((N_SPARSE, bs), jnp.int32),
            pltpu.VMEM((rows, bs), jnp.float32),
            pltpu.SemaphoreType.DMA,
        ],
        compiler_params=pltpu.CompilerParams(use_tc_tiling_on_sc=False),
    )
    def gather(table_hbm, idx_hbm, out_hbm, idx_v, g_v, sem):
        wid = lax.axis_index("s") * 2 + lax.axis_index("c")
        b0 = wid * bs
        pltpu.sync_copy(idx_hbm.at[:, pl.ds(b0, bs)], idx_v)
        for c in range(N_SPARSE // FPC):

            def body(f, _):
                fg = c * FPC + f
                for e in range(EMB):
                    pltpu.async_copy(
                        table_hbm.at[fg * EMB + e].at[idx_v.at[fg]],
                        g_v.at[f * EMB + e], sem)
                @pl.when(f > 0)
                def _():
                    pltpu.make_async_copy(
                        table_hbm.at[pl.ds(0, EMB), pl.ds(0, bs)],
                        g_v.at[pl.ds(0, EMB)], sem).wait()
                return 0

            lax.fori_loop(0, FPC, body, 0)
            pltpu.make_async_copy(
                table_hbm.at[pl.ds(0, EMB), pl.ds(0, bs)],
                g_v.at[pl.ds(0, EMB)], sem).wait()
            pltpu.sync_copy(g_v, out_hbm.at[pl.ds(c * rows, rows),
                                            pl.ds(b0, bs)])

    return gather


def _dense_t_body(embt_ref, dent_ref, cw_ref, cb_ref, w1t_ref,
                  b1_ref, w2t_ref, b2_ref, w3t_ref, b3_ref, wft_ref, bf_ref,
                  out_ref):
    f32 = jnp.float32
    b16 = jnp.bfloat16
    xt = jnp.concatenate([embt_ref[...], dent_ref[...]],
                         axis=0)
    xt16 = xt.astype(b16)
    cw = cw_ref[...]
    xl = xt
    for i in range(LAYER_NUM):
        s = jnp.dot(cw[i:i + 1, :], xl.astype(b16),
                    preferred_element_type=f32)
        xl = (xt * s + cb_ref[i][:, None]) + xl
    ht16 = xt16
    for wt_ref, b_ref in ((w1t_ref, b1_ref), (w2t_ref, b2_ref),
                          (w3t_ref, b3_ref)):
        h = jnp.dot(wt_ref[...], ht16, preferred_element_type=f32)
        ht16 = jnp.maximum(h + b_ref[...], 0.0).astype(b16)
    cat = jnp.concatenate([xl.astype(b16), ht16], axis=0)
    t = jnp.dot(wft_ref[...], cat, preferred_element_type=f32) + bf_ref[...]
    out_ref[...] = jax.nn.sigmoid(t)


def _dense_t_forward(embt, dent, cw2, cb2, w1, b1, w2, b2, w3, b3,
                     wf, bf, blk: int):
    b16 = jnp.bfloat16
    b_total = embt.shape[1]
    h0, h1, h2 = w1.shape[1], w2.shape[1], w3.shape[1]
    full = lambda *shape: pl.BlockSpec(shape, lambda i: (0,) * len(shape))
    return pl.pallas_call(
        _dense_t_body,
        grid=(b_total // blk,),
        in_specs=[
            pl.BlockSpec((NF, blk), lambda i: (0, i)),
            pl.BlockSpec((N_DENSE, blk), lambda i: (0, i)),
            full(LAYER_NUM, DIM),
            full(LAYER_NUM, DIM),
            full(h0, DIM),
            full(h0, 1),
            full(h1, h0),
            full(h1, 1),
            full(h2, h1),
            full(h2, 1),
            full(1, DIM + h2),
            full(1, 1),
        ],
        out_specs=pl.BlockSpec((1, blk), lambda i: (0, i)),
        out_shape=jax.ShapeDtypeStruct((1, b_total), jnp.float32),
        compiler_params=pltpu.CompilerParams(
            dimension_semantics=("arbitrary",)),
    )(embt, dent, cw2.astype(b16), cb2,
      w1.astype(b16).T, b1.reshape(h0, 1),
      w2.astype(b16).T, b2.reshape(h1, 1),
      w3.astype(b16).T, b3.reshape(h2, 1),
      wf.astype(b16).T, bf.reshape(1, 1))


def kernel(inputs, E, cw, cb, W1, b1, W2, b2, W3, b3, Wf, bf):
    b_total = inputs.shape[0]
    int_ = jnp.swapaxes(inputs, 0, 1)
    dent = int_[N_SPARSE:]
    idxt = int_[:N_SPARSE].astype(jnp.int32)
    table = jnp.swapaxes(E, 1, 2).reshape(NF, VOCAB)
    embt = _make_gather_t(b_total)(table, idxt)
    outt = _dense_t_forward(embt, dent, cw[..., 0], cb[..., 0],
                            W1, b1, W2, b2, W3, b3, Wf, bf, blk=2048)
    return outt.reshape(b_total, 1)
